# baseline (speedup 1.0000x reference)
import functools

import jax
import jax.numpy as jnp
from jax import lax
from jax.experimental import pallas as pl
from jax.experimental.pallas import tpu as pltpu
from jax.experimental.pallas import tpu_sc as plsc

N_ROWS = 32
N_COLS = 100000
K = 8
INV_T = 0.5
L = 16
NCH = 10
CH = N_COLS // NCH
U = 5
STEP = U * L
CH_STEPS = CH // STEP
CPL = 256
CAND = CPL * L
BIG_NEG = -3.0e38
I32_MAX = 2**31 - 1

_mesh = plsc.VectorSubcoreMesh(core_axis_name="c", subcore_axis_name="s")


@functools.partial(
    pl.kernel,
    mesh=_mesh,
    compiler_params=pltpu.CompilerParams(
        needs_layout_passes=False, use_tc_tiling_on_sc=False
    ),
    out_type=jax.ShapeDtypeStruct((N_ROWS, N_COLS), jnp.float32),
    scratch_types=[
        pltpu.VMEM((N_COLS,), jnp.float32),
        pltpu.VMEM((CAND,), jnp.float32),
        pltpu.VMEM((CAND,), jnp.int32),
        pltpu.SemaphoreType.DMA((NCH,)),
    ],
)
def _diff_topk_rows(cls_hbm, out_hbm, row, cand_v, cand_i, sems):
    cid = lax.axis_index("c")
    sid = lax.axis_index("s")
    wid = sid * 2 + cid

    copies = [
        pltpu.async_copy(
            cls_hbm.at[wid, pl.ds(c * CH, CH)], row.at[pl.ds(c * CH, CH)],
            sems.at[c],
        )
        for c in range(NCH)
    ]

    lanes = lax.iota(jnp.int32, L)
    neg = jnp.full((L,), BIG_NEG, jnp.float32)
    imax_v = jnp.full((L,), I32_MAX, jnp.int32)
    zf = jnp.zeros((L,), jnp.float32)

    def init_body(i, c):
        cand_v[pl.ds(i * L, L)] = neg
        cand_i[pl.ds(i * L, L)] = imax_v
        return c

    lax.fori_loop(0, CAND // L, init_body, 0)

    m0 = m1 = neg
    a0 = a1 = zf
    for c in range(NCH):
        copies[c].wait()

        def amax_body(i, carry, _base=c * CH):
            m0, m1, a0, a1 = carry
            base = _base + i * STEP
            for u in range(U):
                v = row[pl.ds(base + u * L, L)]
                e = jnp.exp(v * INV_T)
                if u % 2 == 0:
                    m0 = jnp.maximum(m0, v)
                    a0 = a0 + e
                else:
                    m1 = jnp.maximum(m1, v)
                    a1 = a1 + e
            return (m0, m1, a0, a1)

        m0, m1, a0, a1 = lax.fori_loop(0, CH_STEPS, amax_body, (m0, m1, a0, a1))

    m_lane = jnp.maximum(m0, m1)
    s_total = jnp.sum(a0 + a1)
    mv = m_lane
    for _ in range(K - 1):
        cur = jnp.max(mv)
        mv = jnp.where(mv == cur, neg, mv)
    thr = jnp.max(mv)

    lim = CAND - L + lanes

    def collect_body(i, posv):
        base = i * STEP
        for u in range(U):
            off = base + u * L
            v = row[pl.ds(off, L)]
            row[pl.ds(off, L)] = zf
            msk = v >= thr
            pos = jnp.minimum(posv, lim)
            plsc.store_scatter(cand_v, [pos], v, mask=msk)
            plsc.store_scatter(cand_i, [pos], off + lanes, mask=msk)
            posv = posv + jnp.where(msk, L, 0)
        return posv

    posv = lax.fori_loop(0, N_COLS // STEP, collect_body, lanes)
    n_slots = jnp.max(posv - lanes) // L

    chosen_v = []
    chosen_i = []
    for j in range(K):

        def sel_body(c, carry, _chosen_i=tuple(chosen_i)):
            bv, bi = carry
            v = cand_v[pl.ds(c * L, L)]
            ii = cand_i[pl.ds(c * L, L)]
            better = (v > bv) | ((v == bv) & (ii < bi))
            for pj in _chosen_i:
                better = better & (ii != pj)
            bv = jnp.where(better, v, bv)
            bi = jnp.where(better, ii, bi)
            return (bv, bi)

        bv, bi = lax.fori_loop(0, n_slots, sel_body, (neg, imax_v))
        vj = jnp.max(bv)
        ij = jnp.min(jnp.where(bv == vj, bi, imax_v))
        chosen_v.append(vj)
        chosen_i.append(ij)

    v_vec = neg
    i_vec = jnp.zeros((L,), jnp.int32)
    for j in range(K):
        sel = lanes == j
        v_vec = jnp.where(sel, chosen_v[j], v_vec)
        i_vec = jnp.where(sel, chosen_i[j], i_vec)
    e_vec = jnp.exp(v_vec * INV_T)
    d_vec = s_total - (plsc.cumsum(e_vec) - e_vec)
    w_vec = e_vec / d_vec

    plsc.store_scatter(row, [i_vec], w_vec, mask=lanes < K)
    pltpu.sync_copy(row, out_hbm.at[wid])


def kernel(normu, cls):
    classes = _diff_topk_rows(cls)
    return (normu, classes)


# speedup vs baseline: 1.3601x; 1.3601x over previous
import functools

import jax
import jax.numpy as jnp
from jax import lax
from jax.experimental import pallas as pl
from jax.experimental.pallas import tpu as pltpu
from jax.experimental.pallas import tpu_sc as plsc

N_ROWS = 32
N_COLS = 100000
K = 8
INV_T = 0.5
L = 16
NCH = 10
CH = N_COLS // NCH
U = 5
STEP = U * L
CH_STEPS = CH // STEP
CPL = 256
CAND = CPL * L
BIG_NEG = -3.0e38
I32_MAX = 2**31 - 1

_mesh = plsc.VectorSubcoreMesh(core_axis_name="c", subcore_axis_name="s")


@functools.partial(
    pl.kernel,
    mesh=_mesh,
    compiler_params=pltpu.CompilerParams(needs_layout_passes=False),
    out_type=jax.ShapeDtypeStruct((N_ROWS, N_COLS), jnp.float32),
    scratch_types=[
        pltpu.VMEM((N_COLS,), jnp.float32),
        pltpu.VMEM((CAND,), jnp.float32),
        pltpu.VMEM((CAND,), jnp.int32),
        pltpu.SemaphoreType.DMA((NCH,)),
    ],
)
def _diff_topk_rows(cls_hbm, out_hbm, row, cand_v, cand_i, sems):
    cid = lax.axis_index("c")
    sid = lax.axis_index("s")
    wid = sid * 2 + cid

    copy_in = pltpu.async_copy(cls_hbm.at[wid], row, sems.at[0])

    lanes = lax.iota(jnp.int32, L)
    neg = jnp.full((L,), BIG_NEG, jnp.float32)
    imax_v = jnp.full((L,), I32_MAX, jnp.int32)
    zf = jnp.zeros((L,), jnp.float32)

    def init_body(i, c):
        cand_v[pl.ds(i * L, L)] = neg
        cand_i[pl.ds(i * L, L)] = imax_v
        return c

    lax.fori_loop(0, CAND // L, init_body, 0)

    copy_in.wait()

    def amax_body(i, carry):
        m0, m1, a0, a1 = carry
        base = i * STEP
        for u in range(U):
            v = row[pl.ds(base + u * L, L)]
            e = jnp.exp(v * INV_T)
            if u % 2 == 0:
                m0 = jnp.maximum(m0, v)
                a0 = a0 + e
            else:
                m1 = jnp.maximum(m1, v)
                a1 = a1 + e
        return (m0, m1, a0, a1)

    m0, m1, a0, a1 = lax.fori_loop(
        0, N_COLS // STEP, amax_body, (neg, neg, zf, zf)
    )

    m_lane = jnp.maximum(m0, m1)
    s_total = jnp.sum(a0 + a1)
    mv = m_lane
    for _ in range(K - 1):
        cur = jnp.max(mv)
        mv = jnp.where(mv == cur, neg, mv)
    thr = jnp.max(mv)

    lim = CAND - L + lanes

    def collect_body(i, posv):
        base = i * STEP
        for u in range(U):
            off = base + u * L
            v = row[pl.ds(off, L)]
            row[pl.ds(off, L)] = zf
            msk = v >= thr
            pos = jnp.minimum(posv, lim)
            plsc.store_scatter(cand_v, [pos], v, mask=msk)
            plsc.store_scatter(cand_i, [pos], off + lanes, mask=msk)
            posv = posv + jnp.where(msk, L, 0)
        return posv

    posv = lax.fori_loop(0, N_COLS // STEP, collect_body, lanes)
    n_slots = jnp.max(posv - lanes) // L

    chosen_v = []
    chosen_i = []
    for j in range(K):

        def sel_body(c, carry, _chosen_i=tuple(chosen_i)):
            bv, bi = carry
            v = cand_v[pl.ds(c * L, L)]
            ii = cand_i[pl.ds(c * L, L)]
            better = (v > bv) | ((v == bv) & (ii < bi))
            for pj in _chosen_i:
                better = better & (ii != pj)
            bv = jnp.where(better, v, bv)
            bi = jnp.where(better, ii, bi)
            return (bv, bi)

        bv, bi = lax.fori_loop(0, n_slots, sel_body, (neg, imax_v))
        vj = jnp.max(bv)
        ij = jnp.min(jnp.where(bv == vj, bi, imax_v))
        chosen_v.append(vj)
        chosen_i.append(ij)

    v_vec = neg
    i_vec = jnp.zeros((L,), jnp.int32)
    for j in range(K):
        sel = lanes == j
        v_vec = jnp.where(sel, chosen_v[j], v_vec)
        i_vec = jnp.where(sel, chosen_i[j], i_vec)
    e_vec = jnp.exp(v_vec * INV_T)
    d_vec = s_total - (plsc.cumsum(e_vec) - e_vec)
    w_vec = e_vec / d_vec

    plsc.store_scatter(row, [i_vec], w_vec, mask=lanes < K)
    pltpu.sync_copy(row, out_hbm.at[wid])


def kernel(normu, cls):
    classes = _diff_topk_rows(cls)
    return (normu, classes)
